# baseline (speedup 1.0000x reference)
import jax
import jax.numpy as jnp
from jax import lax
from jax.experimental import pallas as pl
from jax.experimental.pallas import tpu as pltpu
from jax.experimental.pallas import tpu_sc as plsc

ROWS = 128
COLS = 32768
FILLER = 32

NS = 16
LANES = 16
RB = 8

SCAN_CH = 512
HEAD = 2048
N_HEAD_CH = HEAD // SCAN_CH
N_BLK = COLS // HEAD
THRESH = 0.5

MEMSET_CH = 4096


def _memset_body(o_ref):
    o_ref[...] = jnp.zeros_like(o_ref)


def _tc_zeros():
    return pl.pallas_call(
        _memset_body,
        out_shape=jax.ShapeDtypeStruct((ROWS, COLS), jnp.float32),
        grid=(COLS // MEMSET_CH,),
        out_specs=pl.BlockSpec((ROWS, MEMSET_CH), lambda i: (0, i)),
    )()


def _sc_body(in_hbm, fp_hbm, h_hbm, cnt_hbm, flag_hbm,
             in_buf, fp_buf, out_buf, zero_buf, cnt_buf, fbuf,
             shared_flag, sem_in, sem_out, sem_misc):
    s = lax.axis_index("s")
    row0 = s * RB

    pltpu.async_copy(
        in_hbm.at[pl.ds(row0, RB), pl.ds(0, SCAN_CH)], in_buf, sem_in)
    pltpu.async_copy(
        fp_hbm.at[pl.ds(row0, RB), pl.ds(0, SCAN_CH)], fp_buf, sem_in)

    def zero_init(i, carry):
        zero_buf[i // (SCAN_CH // LANES),
                 pl.ds((i % (SCAN_CH // LANES)) * LANES, LANES)] = (
            jnp.zeros((LANES,), jnp.float32))
        return carry

    lax.fori_loop(0, RB * SCAN_CH // LANES, zero_init, 0)

    def scan_cond(state):
        ch = state[0]
        cnts = state[1:]
        cnt_min = cnts[0]
        for v in cnts[1:]:
            cnt_min = jnp.minimum(cnt_min, v)
        return jnp.logical_and(cnt_min < FILLER, ch < N_HEAD_CH)

    def scan_body(state):
        ch = state[0]
        cnts = list(state[1:])
        start = pl.multiple_of(ch * SCAN_CH, SCAN_CH)

        @pl.when(ch > 0)
        def _():
            pltpu.async_copy(
                in_hbm.at[pl.ds(row0, RB), pl.ds(start, SCAN_CH)],
                in_buf, sem_in)
            pltpu.async_copy(
                fp_hbm.at[pl.ds(row0, RB), pl.ds(start, SCAN_CH)],
                fp_buf, sem_in)
        pltpu.make_async_copy(
            in_hbm.at[pl.ds(row0, RB), pl.ds(start, SCAN_CH)],
            in_buf, sem_in).wait()
        pltpu.make_async_copy(
            fp_hbm.at[pl.ds(row0, RB), pl.ds(start, SCAN_CH)],
            fp_buf, sem_in).wait()
        for rr in range(RB):
            def vec_body(i, cnt, rr=rr):
                vi = in_buf[rr, pl.ds(i * LANES, LANES)]
                vf = fp_buf[rr, pl.ds(i * LANES, LANES)]
                m = jnp.logical_and(vi > 0.0, vf > THRESH)
                ones = jnp.where(m, jnp.float32(1.0), jnp.float32(0.0))
                cs = plsc.cumsum(ones)
                keep = jnp.logical_and(
                    m, (cnt.astype(jnp.float32) + cs)
                    <= jnp.float32(FILLER))
                out_buf[rr, pl.ds(i * LANES, LANES)] = jnp.where(
                    keep, jnp.float32(1.0), jnp.float32(0.0))
                return cnt + jnp.sum(ones).astype(jnp.int32)
            cnts[rr] = lax.fori_loop(
                0, SCAN_CH // LANES, vec_body, cnts[rr])
        pltpu.async_copy(
            out_buf, h_hbm.at[pl.ds(row0, RB), pl.ds(start, SCAN_CH)],
            sem_out).wait()
        return (ch + 1, *cnts)

    end_state = lax.while_loop(scan_cond, scan_body, (0,) + (0,) * RB)
    ch_end = end_state[0]
    cnts = end_state[1:]

    def fill_fire(j, carry):
        st = pl.multiple_of(j * SCAN_CH, SCAN_CH)

        @pl.when(j >= ch_end)
        def _():
            pltpu.async_copy(
                zero_buf, h_hbm.at[pl.ds(row0, RB), pl.ds(st, SCAN_CH)],
                sem_misc)
        return carry

    lax.fori_loop(0, N_HEAD_CH, fill_fire, 0)

    def fill_drain(j, carry):
        st = pl.multiple_of(j * SCAN_CH, SCAN_CH)

        @pl.when(j >= ch_end)
        def _():
            pltpu.make_async_copy(
                zero_buf, h_hbm.at[pl.ds(row0, RB), pl.ds(st, SCAN_CH)],
                sem_misc).wait()
        return carry

    lax.fori_loop(0, N_HEAD_CH, fill_drain, 0)

    rare = jnp.int32(0)
    for rr in range(RB):
        cnt_buf[rr, pl.ds(0, LANES)] = jnp.full(
            (LANES,), cnts[rr], jnp.int32)
        rare = rare | jnp.where(cnts[rr] < FILLER,
                                jnp.int32(1), jnp.int32(0))
    pltpu.async_copy(
        cnt_buf, cnt_hbm.at[pl.ds(row0, RB), pl.ds(0, 128)],
        sem_misc).wait()

    fbuf[0, pl.ds(0, LANES)] = jnp.full((LANES,), rare, jnp.int32)
    pltpu.async_copy(
        fbuf.at[0], shared_flag.at[s], sem_misc).wait()
    plsc.subcore_barrier()

    @pl.when(s == 0)
    def _aggregate():
        pltpu.async_copy(shared_flag, fbuf, sem_misc).wait()
        f = fbuf[0, pl.ds(0, LANES)]
        for t in range(1, NS):
            f = f | fbuf[t, pl.ds(0, LANES)]
        fbuf[0, pl.ds(0, LANES)] = f
        pltpu.async_copy(fbuf.at[0], flag_hbm, sem_misc).wait()


def _sc_head_scan(inp, fp):
    mesh = plsc.VectorSubcoreMesh(
        core_axis_name="c", subcore_axis_name="s",
        num_cores=1, num_subcores=NS)
    return pl.kernel(
        _sc_body,
        out_type=(
            jax.ShapeDtypeStruct((ROWS, HEAD), jnp.float32),
            jax.ShapeDtypeStruct((ROWS, 128), jnp.int32),
            jax.ShapeDtypeStruct((LANES,), jnp.int32),
        ),
        mesh=mesh,
        scratch_types=[
            pltpu.VMEM((RB, SCAN_CH), jnp.float32),
            pltpu.VMEM((RB, SCAN_CH), jnp.float32),
            pltpu.VMEM((RB, SCAN_CH), jnp.float32),
            pltpu.VMEM((RB, SCAN_CH), jnp.float32),
            pltpu.VMEM((RB, 128), jnp.int32),
            pltpu.VMEM((NS, LANES), jnp.int32),
            pltpu.VMEM_SHARED((NS, LANES), jnp.int32),
            pltpu.SemaphoreType.DMA,
            pltpu.SemaphoreType.DMA,
            pltpu.SemaphoreType.DMA,
        ],
        compiler_params=pltpu.CompilerParams(needs_layout_passes=False),
    )(inp, fp)


def _prefix_mm(v):
    r, c = v.shape
    sub = 128
    n = c // sub
    vr = v.reshape(r * n, sub)
    tri = (lax.broadcasted_iota(jnp.int32, (sub, sub), 0)
           <= lax.broadcasted_iota(jnp.int32, (sub, sub), 1)
           ).astype(jnp.float32)
    s1 = jnp.dot(vr, tri, preferred_element_type=jnp.float32)
    s1 = s1.reshape(r, n, sub)
    tots = s1[:, :, sub - 1]
    trin = (lax.broadcasted_iota(jnp.int32, (n, n), 0)
            < lax.broadcasted_iota(jnp.int32, (n, n), 1)
            ).astype(jnp.float32)
    offs = jnp.dot(tots, trin, preferred_element_type=jnp.float32)
    return (s1 + offs[:, :, None]).reshape(r, c)


def _merge_body(flag_ref, h_ref, cnt_ref, in_ref, fp_ref, z_ref,
                o_ref, carry_ref, in_vmem, fp_vmem, sem):
    rare = flag_ref[0] > 0
    j = pl.program_id(0)

    @pl.when(jnp.logical_not(rare))
    def _():
        o_ref[...] = h_ref[...]

    @pl.when(rare)
    def _():
        @pl.when(j == 0)
        def _():
            o_ref[...] = h_ref[...]
            carry_ref[...] = cnt_ref[:, :1].astype(jnp.float32)

        @pl.when(j > 0)
        def _():
            start = pl.multiple_of(j * HEAD, HEAD)
            pltpu.async_copy(
                in_ref.at[slice(None), pl.ds(start, HEAD)], in_vmem, sem)
            pltpu.async_copy(
                fp_ref.at[slice(None), pl.ds(start, HEAD)], fp_vmem, sem)
            pltpu.make_async_copy(
                in_ref.at[slice(None), pl.ds(start, HEAD)], in_vmem,
                sem).wait()
            pltpu.make_async_copy(
                fp_ref.at[slice(None), pl.ds(start, HEAD)], fp_vmem,
                sem).wait()
            binar = jnp.where(
                jnp.logical_and(in_vmem[...] > 0.0,
                                fp_vmem[...] > THRESH),
                jnp.float32(1.0), jnp.float32(0.0))
            pref = _prefix_mm(binar)
            carry = carry_ref[...]
            keep = jnp.logical_and(
                binar > 0.0, carry + pref <= jnp.float32(FILLER))
            o_ref[...] = jnp.where(keep, jnp.float32(1.0),
                                   jnp.float32(0.0))
            carry_ref[...] = carry + pref[:, -1:]


def _collapsed(j, flag_ref):
    return (0, jnp.where(flag_ref[0] > 0, j, 0))


def _tc_merge(flag, h, cnt, inp, fp, z):
    grid_spec = pltpu.PrefetchScalarGridSpec(
        num_scalar_prefetch=1,
        grid=(N_BLK,),
        in_specs=[
            pl.BlockSpec((ROWS, HEAD), lambda j, f: (0, 0)),
            pl.BlockSpec((ROWS, 128), lambda j, f: (0, 0)),
            pl.BlockSpec(memory_space=pltpu.MemorySpace.HBM),
            pl.BlockSpec(memory_space=pltpu.MemorySpace.HBM),
            pl.BlockSpec((ROWS, HEAD), _collapsed),
        ],
        out_specs=pl.BlockSpec((ROWS, HEAD), _collapsed),
        scratch_shapes=[pltpu.VMEM((ROWS, 1), jnp.float32),
                        pltpu.VMEM((ROWS, HEAD), jnp.float32),
                        pltpu.VMEM((ROWS, HEAD), jnp.float32),
                        pltpu.SemaphoreType.DMA],
    )
    return pl.pallas_call(
        _merge_body,
        grid_spec=grid_spec,
        out_shape=jax.ShapeDtypeStruct((ROWS, COLS), jnp.float32),
        input_output_aliases={5: 0},
    )(flag, h, cnt, inp, fp, z)


@jax.jit
def _fake_profile(inp, fp):
    zeros = _tc_zeros()
    in_head = lax.slice(inp, (0, 0), (ROWS, HEAD))
    fp_head = lax.slice(fp, (0, 0), (ROWS, HEAD))
    h, cnt, flag = _sc_head_scan(in_head, fp_head)
    return _tc_merge(flag, h, cnt, inp, fp, zeros)


def kernel(input, fake_param):
    return _fake_profile(input, fake_param)


# speedup vs baseline: 1.2487x; 1.2487x over previous
import jax
import jax.numpy as jnp
from jax import lax
from jax.experimental import pallas as pl
from jax.experimental.pallas import tpu as pltpu
from jax.experimental.pallas import tpu_sc as plsc

ROWS = 128
COLS = 32768
FILLER = 32

NC = 2
NS = 16
LANES = 16
RB = 8
N_BLOCKS = ROWS // RB

SCAN_CH = 512
N_SCAN_CH = COLS // SCAN_CH
THRESH = 0.5

MEMSET_CH = 4096


def _memset_body(o_ref):
    o_ref[...] = jnp.zeros_like(o_ref)


def _tc_zeros():
    return pl.pallas_call(
        _memset_body,
        out_shape=jax.ShapeDtypeStruct((ROWS, COLS), jnp.float32),
        grid=(COLS // MEMSET_CH,),
        out_specs=pl.BlockSpec((ROWS, MEMSET_CH), lambda i: (0, i)),
    )()


def _sc_update(refs):
    in_hbm, fp_hbm, out_hbm = refs
    mesh = plsc.VectorSubcoreMesh(
        core_axis_name="c", subcore_axis_name="s",
        num_cores=1, num_subcores=NS)

    @pl.core_map(
        mesh,
        compiler_params=pltpu.CompilerParams(needs_layout_passes=False))
    def _():
        c = lax.axis_index("c")
        s = lax.axis_index("s")
        row0 = s * RB

        def scoped(in_buf, fp_buf, out_buf, sem_in, sem_out):
            pltpu.async_copy(
                in_hbm.at[pl.ds(row0, RB), pl.ds(0, SCAN_CH)], in_buf,
                sem_in)
            pltpu.async_copy(
                fp_hbm.at[pl.ds(row0, RB), pl.ds(0, SCAN_CH)], fp_buf,
                sem_in)

            def scan_cond(state):
                ch = state[0]
                cnts = state[1:]
                cnt_min = cnts[0]
                for v in cnts[1:]:
                    cnt_min = jnp.minimum(cnt_min, v)
                return jnp.logical_and(cnt_min < FILLER, ch < N_SCAN_CH)

            def scan_body(state):
                ch = state[0]
                cnts = list(state[1:])
                start = pl.multiple_of(ch * SCAN_CH, SCAN_CH)

                @pl.when(ch > 0)
                def _():
                    pltpu.async_copy(
                        in_hbm.at[pl.ds(row0, RB), pl.ds(start, SCAN_CH)],
                        in_buf, sem_in)
                    pltpu.async_copy(
                        fp_hbm.at[pl.ds(row0, RB), pl.ds(start, SCAN_CH)],
                        fp_buf, sem_in)
                pltpu.make_async_copy(
                    in_hbm.at[pl.ds(row0, RB), pl.ds(start, SCAN_CH)],
                    in_buf, sem_in).wait()
                pltpu.make_async_copy(
                    fp_hbm.at[pl.ds(row0, RB), pl.ds(start, SCAN_CH)],
                    fp_buf, sem_in).wait()
                for rr in range(RB):
                    def vec_body(i, cnt, rr=rr):
                        vi = in_buf[rr, pl.ds(i * LANES, LANES)]
                        vf = fp_buf[rr, pl.ds(i * LANES, LANES)]
                        m = jnp.logical_and(vi > 0.0, vf > THRESH)
                        ones = jnp.where(m, jnp.float32(1.0),
                                         jnp.float32(0.0))
                        cs = plsc.cumsum(ones)
                        keep = jnp.logical_and(
                            m, (cnt.astype(jnp.float32) + cs)
                            <= jnp.float32(FILLER))
                        out_buf[rr, pl.ds(i * LANES, LANES)] = jnp.where(
                            keep, jnp.float32(1.0), jnp.float32(0.0))
                        return cnt + jnp.sum(ones).astype(jnp.int32)
                    cnts[rr] = lax.fori_loop(
                        0, SCAN_CH // LANES, vec_body, cnts[rr])
                pltpu.async_copy(
                    out_buf,
                    out_hbm.at[pl.ds(row0, RB), pl.ds(start, SCAN_CH)],
                    sem_out).wait()
                return (ch + 1, *cnts)

            lax.while_loop(scan_cond, scan_body, (0,) + (0,) * RB)

        @pl.when(c == 0)
        def _owner():
            pl.run_scoped(
                scoped,
                pltpu.VMEM((RB, SCAN_CH), jnp.float32),
                pltpu.VMEM((RB, SCAN_CH), jnp.float32),
                pltpu.VMEM((RB, SCAN_CH), jnp.float32),
                pltpu.SemaphoreType.DMA,
                pltpu.SemaphoreType.DMA,
            )


@jax.jit
def _fake_profile(inp, fp):
    zeros = _tc_zeros()
    _, _, out = pl.run_state(_sc_update)((inp, fp, zeros))
    return out


def kernel(input, fake_param):
    return _fake_profile(input, fake_param)


# speedup vs baseline: 1.2914x; 1.0342x over previous
import jax
import jax.numpy as jnp
from jax import lax
from jax.experimental import pallas as pl
from jax.experimental.pallas import tpu as pltpu
from jax.experimental.pallas import tpu_sc as plsc

ROWS = 128
COLS = 32768
FILLER = 32

NC = 2
NS = 16
LANES = 16
RB = 8
N_BLOCKS = ROWS // RB

SCAN_CH = 256
N_SCAN_CH = COLS // SCAN_CH
THRESH = 0.5

MEMSET_CH = 4096


def _memset_body(o_ref):
    o_ref[...] = jnp.zeros_like(o_ref)


def _tc_zeros():
    return pl.pallas_call(
        _memset_body,
        out_shape=jax.ShapeDtypeStruct((ROWS, COLS), jnp.float32),
        grid=(COLS // MEMSET_CH,),
        out_specs=pl.BlockSpec((ROWS, MEMSET_CH), lambda i: (0, i)),
    )()


def _sc_update(refs):
    in_hbm, fp_hbm, out_hbm = refs
    mesh = plsc.VectorSubcoreMesh(
        core_axis_name="c", subcore_axis_name="s",
        num_cores=1, num_subcores=NS)

    @pl.core_map(
        mesh,
        compiler_params=pltpu.CompilerParams(needs_layout_passes=False))
    def _():
        c = lax.axis_index("c")
        s = lax.axis_index("s")
        row0 = s * RB

        def scoped(in_buf, fp_buf, out_buf, sem_in, sem_out):
            pltpu.async_copy(
                in_hbm.at[pl.ds(row0, RB), pl.ds(0, SCAN_CH)], in_buf,
                sem_in)
            pltpu.async_copy(
                fp_hbm.at[pl.ds(row0, RB), pl.ds(0, SCAN_CH)], fp_buf,
                sem_in)

            def scan_cond(state):
                ch = state[0]
                cnts = state[1:]
                cnt_min = cnts[0]
                for v in cnts[1:]:
                    cnt_min = jnp.minimum(cnt_min, v)
                return jnp.logical_and(cnt_min < FILLER, ch < N_SCAN_CH)

            def scan_body(state):
                ch = state[0]
                cnts = list(state[1:])
                start = pl.multiple_of(ch * SCAN_CH, SCAN_CH)

                @pl.when(ch > 0)
                def _():
                    pltpu.async_copy(
                        in_hbm.at[pl.ds(row0, RB), pl.ds(start, SCAN_CH)],
                        in_buf, sem_in)
                    pltpu.async_copy(
                        fp_hbm.at[pl.ds(row0, RB), pl.ds(start, SCAN_CH)],
                        fp_buf, sem_in)
                pltpu.make_async_copy(
                    in_hbm.at[pl.ds(row0, RB), pl.ds(start, SCAN_CH)],
                    in_buf, sem_in).wait()
                pltpu.make_async_copy(
                    fp_hbm.at[pl.ds(row0, RB), pl.ds(start, SCAN_CH)],
                    fp_buf, sem_in).wait()
                for rr in range(RB):
                    def vec_body(i, cnt, rr=rr):
                        vi = in_buf[rr, pl.ds(i * LANES, LANES)]
                        vf = fp_buf[rr, pl.ds(i * LANES, LANES)]
                        m = jnp.logical_and(vi > 0.0, vf > THRESH)
                        ones = jnp.where(m, jnp.float32(1.0),
                                         jnp.float32(0.0))
                        cs = plsc.cumsum(ones)
                        keep = jnp.logical_and(
                            m, (cnt.astype(jnp.float32) + cs)
                            <= jnp.float32(FILLER))
                        out_buf[rr, pl.ds(i * LANES, LANES)] = jnp.where(
                            keep, jnp.float32(1.0), jnp.float32(0.0))
                        return cnt + jnp.sum(ones).astype(jnp.int32)
                    cnts[rr] = lax.fori_loop(
                        0, SCAN_CH // LANES, vec_body, cnts[rr])
                pltpu.async_copy(
                    out_buf,
                    out_hbm.at[pl.ds(row0, RB), pl.ds(start, SCAN_CH)],
                    sem_out).wait()
                return (ch + 1, *cnts)

            lax.while_loop(scan_cond, scan_body, (0,) + (0,) * RB)

        @pl.when(c == 0)
        def _owner():
            pl.run_scoped(
                scoped,
                pltpu.VMEM((RB, SCAN_CH), jnp.float32),
                pltpu.VMEM((RB, SCAN_CH), jnp.float32),
                pltpu.VMEM((RB, SCAN_CH), jnp.float32),
                pltpu.SemaphoreType.DMA,
                pltpu.SemaphoreType.DMA,
            )


@jax.jit
def _fake_profile(inp, fp):
    zeros = _tc_zeros()
    _, _, out = pl.run_state(_sc_update)((inp, fp, zeros))
    return out


def kernel(input, fake_param):
    return _fake_profile(input, fake_param)


# speedup vs baseline: 1.2944x; 1.0023x over previous
import jax
import jax.numpy as jnp
from jax import lax
from jax.experimental import pallas as pl
from jax.experimental.pallas import tpu as pltpu
from jax.experimental.pallas import tpu_sc as plsc

ROWS = 128
COLS = 32768
FILLER = 32

NC = 2
NS = 16
LANES = 16
RB = 8
N_BLOCKS = ROWS // RB

SCAN_CH = 128
N_SCAN_CH = COLS // SCAN_CH
THRESH = 0.5

MEMSET_CH = 4096


def _memset_body(o_ref):
    o_ref[...] = jnp.zeros_like(o_ref)


def _tc_zeros():
    return pl.pallas_call(
        _memset_body,
        out_shape=jax.ShapeDtypeStruct((ROWS, COLS), jnp.float32),
        grid=(COLS // MEMSET_CH,),
        out_specs=pl.BlockSpec((ROWS, MEMSET_CH), lambda i: (0, i)),
    )()


def _sc_update(refs):
    in_hbm, fp_hbm, out_hbm = refs
    mesh = plsc.VectorSubcoreMesh(
        core_axis_name="c", subcore_axis_name="s",
        num_cores=1, num_subcores=NS)

    @pl.core_map(
        mesh,
        compiler_params=pltpu.CompilerParams(needs_layout_passes=False))
    def _():
        c = lax.axis_index("c")
        s = lax.axis_index("s")
        row0 = s * RB

        def scoped(in_buf, fp_buf, out_buf, sem_in, sem_out):
            pltpu.async_copy(
                in_hbm.at[pl.ds(row0, RB), pl.ds(0, SCAN_CH)], in_buf,
                sem_in)
            pltpu.async_copy(
                fp_hbm.at[pl.ds(row0, RB), pl.ds(0, SCAN_CH)], fp_buf,
                sem_in)

            def scan_cond(state):
                ch = state[0]
                cnts = state[1:]
                cnt_min = cnts[0]
                for v in cnts[1:]:
                    cnt_min = jnp.minimum(cnt_min, v)
                return jnp.logical_and(cnt_min < FILLER, ch < N_SCAN_CH)

            def scan_body(state):
                ch = state[0]
                cnts = list(state[1:])
                start = pl.multiple_of(ch * SCAN_CH, SCAN_CH)

                @pl.when(ch > 0)
                def _():
                    pltpu.async_copy(
                        in_hbm.at[pl.ds(row0, RB), pl.ds(start, SCAN_CH)],
                        in_buf, sem_in)
                    pltpu.async_copy(
                        fp_hbm.at[pl.ds(row0, RB), pl.ds(start, SCAN_CH)],
                        fp_buf, sem_in)
                pltpu.make_async_copy(
                    in_hbm.at[pl.ds(row0, RB), pl.ds(start, SCAN_CH)],
                    in_buf, sem_in).wait()
                pltpu.make_async_copy(
                    fp_hbm.at[pl.ds(row0, RB), pl.ds(start, SCAN_CH)],
                    fp_buf, sem_in).wait()
                for rr in range(RB):
                    def vec_body(i, cnt, rr=rr):
                        vi = in_buf[rr, pl.ds(i * LANES, LANES)]
                        vf = fp_buf[rr, pl.ds(i * LANES, LANES)]
                        m = jnp.logical_and(vi > 0.0, vf > THRESH)
                        ones = jnp.where(m, jnp.float32(1.0),
                                         jnp.float32(0.0))
                        cs = plsc.cumsum(ones)
                        keep = jnp.logical_and(
                            m, (cnt.astype(jnp.float32) + cs)
                            <= jnp.float32(FILLER))
                        out_buf[rr, pl.ds(i * LANES, LANES)] = jnp.where(
                            keep, jnp.float32(1.0), jnp.float32(0.0))
                        return cnt + jnp.sum(ones).astype(jnp.int32)
                    cnts[rr] = lax.fori_loop(
                        0, SCAN_CH // LANES, vec_body, cnts[rr])
                pltpu.async_copy(
                    out_buf,
                    out_hbm.at[pl.ds(row0, RB), pl.ds(start, SCAN_CH)],
                    sem_out).wait()
                return (ch + 1, *cnts)

            lax.while_loop(scan_cond, scan_body, (0,) + (0,) * RB)

        @pl.when(c == 0)
        def _owner():
            pl.run_scoped(
                scoped,
                pltpu.VMEM((RB, SCAN_CH), jnp.float32),
                pltpu.VMEM((RB, SCAN_CH), jnp.float32),
                pltpu.VMEM((RB, SCAN_CH), jnp.float32),
                pltpu.SemaphoreType.DMA,
                pltpu.SemaphoreType.DMA,
            )


@jax.jit
def _fake_profile(inp, fp):
    zeros = _tc_zeros()
    _, _, out = pl.run_state(_sc_update)((inp, fp, zeros))
    return out


def kernel(input, fake_param):
    return _fake_profile(input, fake_param)
